# baseline (speedup 1.0000x reference)
import functools

import jax
import jax.numpy as jnp
from jax import lax
from jax.experimental import pallas as pl
from jax.experimental.pallas import tpu as pltpu
from jax.experimental.pallas import tpu_sc as plsc

N = 10000
E = 160000
D = 256
H = 512

ROWS = 1000

EB = 128
NT = 16
TB = 84
EPAD = NT * TB * EB
EROWS = EPAD // EB
NPAD1 = 10016
RT1 = NPAD1 // NT
NPAD2 = 10240
RT2 = NPAD2 // NT
NB2 = 4

_MESH = plsc.VectorSubcoreMesh(core_axis_name="c", subcore_axis_name="s")


def _dot_t(a, b):
    return lax.dot_general(a, b, (((1,), (1,)), ((), ())),
                           preferred_element_type=jnp.float32)


def _stage1_body(x_ref, wp1_ref, bp1_ref, wr1_ref, h1_ref, xr1_ref):
    xb = x_ref[...]
    h1_ref[...] = jnp.maximum(_dot_t(xb, wp1_ref[...]) + bp1_ref[...], 0.0)
    xr1_ref[...] = _dot_t(xb, wr1_ref[...])


def _stage1(x, Wp1, bp1, Wr1):
    return pl.pallas_call(
        _stage1_body,
        grid=(N // ROWS,),
        in_specs=[
            pl.BlockSpec((ROWS, D), lambda i: (i, 0)),
            pl.BlockSpec((D, D), lambda i: (0, 0)),
            pl.BlockSpec((1, D), lambda i: (0, 0)),
            pl.BlockSpec((H, D), lambda i: (0, 0)),
        ],
        out_specs=[
            pl.BlockSpec((ROWS, D), lambda i: (i, 0)),
            pl.BlockSpec((ROWS, H), lambda i: (i, 0)),
        ],
        out_shape=[
            jax.ShapeDtypeStruct((N, D), jnp.float32),
            jax.ShapeDtypeStruct((N, H), jnp.float32),
        ],
    )(x, Wp1, bp1.reshape(1, D), Wr1)


def _deg_body(dst_hbm, cnt_hbm, ones_v, zc, dst3, cnt_sh, sem_c, sem_z):
    c = lax.axis_index("c")
    s = lax.axis_index("s")
    zero16 = jnp.zeros((16,), jnp.float32)
    half = TB // 2
    w = c * NT + s

    def _ones(j, _):
        ones_v[pl.ds(j * 16, 16)] = zero16 + 1.0
        return 0
    lax.fori_loop(0, EB // 16, _ones, 0)

    def _zc(j, _):
        zc[pl.ds(j * 16, 16)] = zero16
        return 0
    lax.fori_loop(0, RT2 // 16, _zc, 0)

    pltpu.async_copy(dst_hbm.at[w, :, :], dst3, sem_z)
    pltpu.sync_copy(zc, cnt_sh.at[pl.ds(s * RT2, RT2)])
    pltpu.make_async_copy(dst_hbm.at[w, :, :], dst3, sem_z).wait()
    plsc.subcore_barrier()

    def _wait_scat(g, slot):
        pltpu.make_async_copy(ones_v, cnt_sh.at[dst3.at[g]],
                              sem_c.at[slot]).wait()

    def _outer(oo, _):
        for j in range(3):
            g = oo * 3 + j

            def _lag():
                _wait_scat(g - 2, (j + 1) % 3)
            if j < 2:
                @pl.when(g >= 2)
                def _():
                    _lag()
            else:
                _lag()
            pltpu.async_copy(ones_v, cnt_sh.at[dst3.at[g]],
                             sem_c.at[j % 3], add=True)
        return 0
    lax.fori_loop(0, half // 3, _outer, 0)
    _wait_scat(jnp.int32(half - 2), (half - 2) % 3)
    _wait_scat(jnp.int32(half - 1), (half - 1) % 3)
    plsc.subcore_barrier()
    pltpu.sync_copy(cnt_sh.at[pl.ds(s * RT2, RT2)],
                    cnt_hbm.at[c, pl.ds(s * RT2, RT2)])


_deg = functools.partial(
    pl.kernel, _deg_body, mesh=_MESH,
    out_type=jax.ShapeDtypeStruct((2, NPAD2), jnp.float32),
    scratch_types=[
        pltpu.VMEM((EB,), jnp.float32),
        pltpu.VMEM((RT2,), jnp.float32),
        pltpu.VMEM((TB // 2, EB), jnp.int32),
        pltpu.VMEM_SHARED((NPAD2,), jnp.float32),
        pltpu.SemaphoreType.DMA((3,)),
        pltpu.SemaphoreType.DMA,
    ],
)()


def _agg1_body(h_hbm, idx_hbm, ssum_hbm,
               rows0, rows1, rows2, i0, i1, i2, i3,
               acc_sh, sem_i, sem_g, sem_s, sem_z):
    c = lax.axis_index("c")
    s = lax.axis_index("s")
    rows = (rows0, rows1, rows2)
    ibs = (i0, i1, i2, i3)
    zero16 = jnp.zeros((16,), jnp.float32)

    def _zrow(r, _):
        def _zcol(j, _):
            rows0[r, pl.ds(j * 16, 16)] = zero16
            return 0
        return lax.fori_loop(0, 8, _zcol, 0)
    lax.fori_loop(0, EB, _zrow, 0)

    def _idx_start(g, slot):
        pltpu.async_copy(idx_hbm.at[c, s * TB + g, :, :], ibs[slot],
                         sem_i.at[slot])

    def _idx_wait(g, slot):
        pltpu.make_async_copy(idx_hbm.at[c, s * TB + g, :, :], ibs[slot],
                              sem_i.at[slot]).wait()

    def _gather_start(g_slot4, slot3):
        pltpu.async_copy(h_hbm.at[ibs[g_slot4].at[0]], rows[slot3],
                         sem_g.at[slot3])

    def _gather_wait(g_slot4, slot3):
        pltpu.make_async_copy(h_hbm.at[ibs[g_slot4].at[0]], rows[slot3],
                              sem_g.at[slot3]).wait()

    def _scat_start(slot4, slot3):
        pltpu.async_copy(rows[slot3], acc_sh.at[ibs[slot4].at[1]],
                         sem_s.at[slot3], add=True)

    def _scat_wait(slot4, slot3):
        pltpu.make_async_copy(rows[slot3], acc_sh.at[ibs[slot4].at[1]],
                              sem_s.at[slot3]).wait()

    for b in range(4):
        _idx_start(jnp.int32(b), b)

    for k in range(4):
        pltpu.async_copy(rows0, acc_sh.at[pl.ds(s * RT1 + k * 128, 128), :],
                         sem_z)
    pltpu.async_copy(rows0.at[pl.ds(0, RT1 - 512), :],
                     acc_sh.at[pl.ds(s * RT1 + 512, RT1 - 512), :], sem_z)
    for k in range(4):
        pltpu.make_async_copy(rows0, acc_sh.at[pl.ds(s * RT1, 128), :],
                              sem_z).wait()
    pltpu.make_async_copy(rows0.at[pl.ds(0, RT1 - 512), :],
                          acc_sh.at[pl.ds(s * RT1, RT1 - 512), :],
                          sem_z).wait()
    _idx_wait(jnp.int32(0), 0)
    _gather_start(0, 0)
    _idx_wait(jnp.int32(1), 1)
    _gather_start(1, 1)
    _idx_wait(jnp.int32(2), 2)
    _gather_start(2, 2)
    plsc.subcore_barrier()

    def _outer(oo, _):
        for j in range(12):
            g = oo * 12 + j
            s3 = j % 3
            s4 = j % 4
            _gather_wait(s4, s3)
            _scat_start(s4, s3)

            def _lag():
                _scat_wait((j + 2) % 4, (j + 1) % 3)

            def _pref():
                _idx_wait(g + 1, (j + 1) % 4)
                _gather_start((j + 1) % 4, (j + 1) % 3)

            def _inext():
                _idx_start(g + 2, (j + 2) % 4)

            if j < 2:
                @pl.when(g >= 2)
                def _():
                    _lag()
                    _pref()
                    _inext()
            elif j == 10:
                _lag()
                _pref()

                @pl.when(g + 2 < TB)
                def _():
                    _inext()
            elif j == 11:
                _lag()

                @pl.when(g + 1 < TB)
                def _():
                    _pref()
                    _inext()
            else:
                _lag()
                _pref()
                _inext()
        return 0
    lax.fori_loop(0, TB // 12, _outer, 0)
    _scat_wait(2, 1)
    _scat_wait(3, 2)
    plsc.subcore_barrier()

    pltpu.sync_copy(acc_sh.at[pl.ds(s * RT1, RT1), :],
                    ssum_hbm.at[c, s, :, :])


_agg1 = functools.partial(
    pl.kernel, _agg1_body, mesh=_MESH,
    out_type=jax.ShapeDtypeStruct((2, NT, RT1, 128), jnp.float32),
    scratch_types=[
        pltpu.VMEM((EB, 128), jnp.float32),
        pltpu.VMEM((EB, 128), jnp.float32),
        pltpu.VMEM((EB, 128), jnp.float32),
        pltpu.VMEM((2, EB), jnp.int32),
        pltpu.VMEM((2, EB), jnp.int32),
        pltpu.VMEM((2, EB), jnp.int32),
        pltpu.VMEM((2, EB), jnp.int32),
        pltpu.VMEM_SHARED((NPAD1, 128), jnp.float32),
        pltpu.SemaphoreType.DMA((4,)),
        pltpu.SemaphoreType.DMA((3,)),
        pltpu.SemaphoreType.DMA((3,)),
        pltpu.SemaphoreType.DMA,
    ],
)()


def _stage2_body(sa_ref, sb_ref, c0_ref, c1_ref, xr1_ref, wl1a_ref,
                 wl1b_ref, bl1_ref, wp2_ref, bp2_ref, wl2_ref, wr2_ref,
                 s2_ref, r2_ref, cs_ref):
    csum = c0_ref[...] + c1_ref[...]
    cs_ref[...] = csum
    inv = 1.0 / jnp.maximum(csum, 1.0)
    lsum = _dot_t(sa_ref[...], wl1a_ref[...]) + _dot_t(sb_ref[...],
                                                       wl1b_ref[...])
    out1 = jnp.maximum(lsum * inv + bl1_ref[...] + xr1_ref[...], 0.0)
    h2 = jnp.maximum(_dot_t(out1, wp2_ref[...]) + bp2_ref[...], 0.0)
    s2_ref[...] = _dot_t(h2, wl2_ref[...])
    r2_ref[...] = _dot_t(out1, wr2_ref[...])


def _stage2(sa, sb, c0, c1, xr1, Wl1, bl1, Wp2, bp2, Wl2, Wr2):
    return pl.pallas_call(
        _stage2_body,
        grid=(N // ROWS,),
        in_specs=[
            pl.BlockSpec((ROWS, 128), lambda i: (i, 0)),
            pl.BlockSpec((ROWS, 128), lambda i: (i, 0)),
            pl.BlockSpec((ROWS, 1), lambda i: (i, 0)),
            pl.BlockSpec((ROWS, 1), lambda i: (i, 0)),
            pl.BlockSpec((ROWS, H), lambda i: (i, 0)),
            pl.BlockSpec((H, 128), lambda i: (0, 0)),
            pl.BlockSpec((H, 128), lambda i: (0, 0)),
            pl.BlockSpec((1, H), lambda i: (0, 0)),
            pl.BlockSpec((H, H), lambda i: (0, 0)),
            pl.BlockSpec((1, H), lambda i: (0, 0)),
            pl.BlockSpec((1, H), lambda i: (0, 0)),
            pl.BlockSpec((1, H), lambda i: (0, 0)),
        ],
        out_specs=[
            pl.BlockSpec((ROWS, 1), lambda i: (i, 0)),
            pl.BlockSpec((ROWS, 1), lambda i: (i, 0)),
            pl.BlockSpec((ROWS, 1), lambda i: (i, 0)),
        ],
        out_shape=[
            jax.ShapeDtypeStruct((N, 1), jnp.float32),
            jax.ShapeDtypeStruct((N, 1), jnp.float32),
            jax.ShapeDtypeStruct((N, 1), jnp.float32),
        ],
    )(sa, sb, c0.reshape(N, 1), c1.reshape(N, 1), xr1, Wl1[:, :128],
      Wl1[:, 128:], bl1.reshape(1, H), Wp2, bp2.reshape(1, H), Wl2, Wr2)


def _agg2_body(s2_hbm, src_hbm, dst_hbm, cnt_hbm, r2_hbm, out_hbm,
               zcnt, s2_v, src3, dst3, vals0, vals1, vals2, vals3,
               a_v, c_v, r_v, o_v, acc_sh, sem_g, sem_s):
    c = lax.axis_index("c")
    s = lax.axis_index("s")
    vals = (vals0, vals1, vals2, vals3)
    zero16 = jnp.zeros((16,), jnp.float32)

    @pl.when(c == 0)
    def _():
        def _zc(j, _):
            zcnt[pl.ds(j * 16, 16)] = zero16
            return 0
        lax.fori_loop(0, RT2 // 16, _zc, 0)
        pltpu.async_copy(src_hbm.at[s, :, :], src3, sem_g.at[0])
        pltpu.async_copy(dst_hbm.at[s, :, :], dst3, sem_g.at[1])
        pltpu.async_copy(s2_hbm, s2_v, sem_g.at[2])
        pltpu.sync_copy(zcnt, acc_sh.at[pl.ds(s * RT2, RT2)])
        pltpu.make_async_copy(src_hbm.at[s, :, :], src3,
                              sem_g.at[0]).wait()
        pltpu.make_async_copy(dst_hbm.at[s, :, :], dst3,
                              sem_g.at[1]).wait()
        pltpu.make_async_copy(s2_hbm, s2_v, sem_g.at[2]).wait()
    plsc.subcore_barrier()

    @pl.when(c == 0)
    def _():
        def _scat_wait(g, slot):
            pltpu.make_async_copy(vals[slot], acc_sh.at[dst3.at[g]],
                                  sem_s.at[slot]).wait()

        def _group(o, _):
            for i in range(NB2):
                g = o * NB2 + i

                @pl.when(g >= NB2)
                def _():
                    _scat_wait(g, i)

                def _gather(j, _):
                    idx16 = src3[g, pl.ds(j * 16, 16)]
                    row16 = lax.shift_right_logical(idx16, 7)
                    col16 = lax.bitwise_and(idx16, 127)
                    vals[i][pl.ds(j * 16, 16)] = plsc.load_gather(
                        s2_v, [row16, col16])
                    return 0
                lax.fori_loop(0, EB // 16, _gather, 0)
                pltpu.async_copy(vals[i], acc_sh.at[dst3.at[g]],
                                 sem_s.at[i], add=True)
            return 0
        lax.fori_loop(0, TB // NB2, _group, 0)
        for i in range(NB2):
            _scat_wait(jnp.int32(TB - NB2 + i), i)
    plsc.subcore_barrier()

    @pl.when(c == 0)
    def _():
        pltpu.sync_copy(acc_sh.at[pl.ds(s * RT2, RT2)], a_v)
        pltpu.sync_copy(cnt_hbm.at[pl.ds(s * RT2, RT2)], c_v)
        pltpu.sync_copy(r2_hbm.at[pl.ds(s * RT2, RT2)], r_v)

        def _fin(j, _):
            sl = pl.ds(j * 16, 16)
            z = a_v[sl] / jnp.maximum(c_v[sl], 1.0) + r_v[sl]
            o_v[sl] = 1.0 / (1.0 + jnp.exp(-z))
            return 0
        lax.fori_loop(0, RT2 // 16, _fin, 0)
        pltpu.sync_copy(o_v, out_hbm.at[pl.ds(s * RT2, RT2)])


_agg2 = functools.partial(
    pl.kernel, _agg2_body, mesh=_MESH,
    compiler_params=pltpu.CompilerParams(needs_layout_passes=False),
    out_type=jax.ShapeDtypeStruct((NPAD2,), jnp.float32),
    scratch_types=[
        pltpu.VMEM((RT2,), jnp.float32),
        pltpu.VMEM((NPAD2 // 128, 128), jnp.float32),
        pltpu.VMEM((TB, EB), jnp.int32),
        pltpu.VMEM((TB, EB), jnp.int32),
        pltpu.VMEM((EB,), jnp.float32),
        pltpu.VMEM((EB,), jnp.float32),
        pltpu.VMEM((EB,), jnp.float32),
        pltpu.VMEM((EB,), jnp.float32),
        pltpu.VMEM((RT2,), jnp.float32),
        pltpu.VMEM((RT2,), jnp.float32),
        pltpu.VMEM((RT2,), jnp.float32),
        pltpu.VMEM((RT2,), jnp.float32),
        pltpu.VMEM_SHARED((NPAD2,), jnp.float32),
        pltpu.SemaphoreType.DMA((NB2,)),
        pltpu.SemaphoreType.DMA((NB2,)),
    ],
)()


def kernel(x, edge_index, Wp1, bp1, Wl1, bl1, Wr1, Wp2, bp2, Wl2, bl2, Wr2):
    src = edge_index[0]
    dst = edge_index[1]
    pad = EPAD - E
    src_p = jnp.concatenate([src, jnp.zeros((pad,), jnp.int32)])
    dst_p = jnp.concatenate([dst, jnp.full((pad,), N, jnp.int32)])
    src2 = src_p.reshape(EROWS, EB)
    dst2 = dst_p.reshape(EROWS, EB)
    idxcat = jnp.stack([
        jnp.stack([src2 * 2, dst2], axis=1),
        jnp.stack([src2 * 2 + 1, dst2], axis=1),
    ])

    cnt2 = _deg(dst2.reshape(2 * NT, TB // 2, EB))
    h1, xr1 = _stage1(x, Wp1, bp1, Wr1)
    ssum = _agg1(h1.reshape(2 * N, 128), idxcat).reshape(2, NPAD1, 128)
    s2, r2, csum = _stage2(ssum[0, :N, :], ssum[1, :N, :], cnt2[0, :N],
                           cnt2[1, :N], xr1, Wl1, bl1, Wp2, bp2, Wl2, Wr2)
    zpad = jnp.zeros((NPAD2 - N,), jnp.float32)
    s2_p = jnp.concatenate([s2.reshape(N), zpad])
    r2_p = jnp.concatenate([r2.reshape(N) + bl2[0], zpad])
    cs_p = jnp.concatenate([csum.reshape(N), zpad])
    src3d = src2.reshape(NT, TB, EB)
    dst3d = dst2.reshape(NT, TB, EB)
    out = _agg2(s2_p.reshape(NPAD2 // 128, 128), src3d, dst3d, cs_p, r2_p)
    return out[:N].reshape(N, 1)


# speedup vs baseline: 1.0158x; 1.0158x over previous
import functools

import jax
import jax.numpy as jnp
from jax import lax
from jax.experimental import pallas as pl
from jax.experimental.pallas import tpu as pltpu
from jax.experimental.pallas import tpu_sc as plsc

N = 10000
E = 160000
D = 256
H = 512

ROWS = 1000

EB = 128
NT = 16
TB = 84
EPAD = NT * TB * EB
EROWS = EPAD // EB
NPAD1 = 10016
RT1 = NPAD1 // NT
NPAD2 = 10240
RT2 = NPAD2 // NT
NB2 = 4

_MESH = plsc.VectorSubcoreMesh(core_axis_name="c", subcore_axis_name="s")


def _dot_t(a, b):
    return lax.dot_general(a, b, (((1,), (1,)), ((), ())),
                           preferred_element_type=jnp.float32)


def _stage1_body(x_ref, wp1_ref, bp1_ref, wr1_ref, h1_ref, xr1_ref):
    xb = x_ref[...]
    h1_ref[...] = jnp.maximum(_dot_t(xb, wp1_ref[...]) + bp1_ref[...], 0.0)
    xr1_ref[...] = _dot_t(xb, wr1_ref[...])


def _stage1(x, Wp1, bp1, Wr1):
    return pl.pallas_call(
        _stage1_body,
        grid=(N // ROWS,),
        in_specs=[
            pl.BlockSpec((ROWS, D), lambda i: (i, 0)),
            pl.BlockSpec((D, D), lambda i: (0, 0)),
            pl.BlockSpec((1, D), lambda i: (0, 0)),
            pl.BlockSpec((H, D), lambda i: (0, 0)),
        ],
        out_specs=[
            pl.BlockSpec((ROWS, D), lambda i: (i, 0)),
            pl.BlockSpec((ROWS, H), lambda i: (i, 0)),
        ],
        out_shape=[
            jax.ShapeDtypeStruct((N, D), jnp.float32),
            jax.ShapeDtypeStruct((N, H), jnp.float32),
        ],
    )(x, Wp1, bp1.reshape(1, D), Wr1)


def _deg_body(dst_hbm, cnt_hbm, ones_v, zc, dst3, cnt_sh, sem_c, sem_z):
    c = lax.axis_index("c")
    s = lax.axis_index("s")
    zero16 = jnp.zeros((16,), jnp.float32)
    half = TB // 2
    w = c * NT + s

    def _ones(j, _):
        ones_v[pl.ds(j * 16, 16)] = zero16 + 1.0
        return 0
    lax.fori_loop(0, EB // 16, _ones, 0)

    def _zc(j, _):
        zc[pl.ds(j * 16, 16)] = zero16
        return 0
    lax.fori_loop(0, RT2 // 16, _zc, 0)

    pltpu.async_copy(dst_hbm.at[w, :, :], dst3, sem_z)
    pltpu.sync_copy(zc, cnt_sh.at[pl.ds(s * RT2, RT2)])
    pltpu.make_async_copy(dst_hbm.at[w, :, :], dst3, sem_z).wait()
    plsc.subcore_barrier()

    def _wait_scat(g, slot):
        pltpu.make_async_copy(ones_v, cnt_sh.at[dst3.at[g]],
                              sem_c.at[slot]).wait()

    def _outer(oo, _):
        for j in range(3):
            g = oo * 3 + j

            def _lag():
                _wait_scat(g - 2, (j + 1) % 3)
            if j < 2:
                @pl.when(g >= 2)
                def _():
                    _lag()
            else:
                _lag()
            pltpu.async_copy(ones_v, cnt_sh.at[dst3.at[g]],
                             sem_c.at[j % 3], add=True)
        return 0
    lax.fori_loop(0, half // 3, _outer, 0)
    _wait_scat(jnp.int32(half - 2), (half - 2) % 3)
    _wait_scat(jnp.int32(half - 1), (half - 1) % 3)
    plsc.subcore_barrier()
    pltpu.sync_copy(cnt_sh.at[pl.ds(s * RT2, RT2)],
                    cnt_hbm.at[c, pl.ds(s * RT2, RT2)])


_deg = functools.partial(
    pl.kernel, _deg_body, mesh=_MESH,
    out_type=jax.ShapeDtypeStruct((2, NPAD2), jnp.float32),
    scratch_types=[
        pltpu.VMEM((EB,), jnp.float32),
        pltpu.VMEM((RT2,), jnp.float32),
        pltpu.VMEM((TB // 2, EB), jnp.int32),
        pltpu.VMEM_SHARED((NPAD2,), jnp.float32),
        pltpu.SemaphoreType.DMA((3,)),
        pltpu.SemaphoreType.DMA,
    ],
)()


G3 = 3
NG1 = TB // G3


def _agg1_body(h_hbm, idx_hbm, ssum_hbm,
               rows0, rows1, rows2, ib0, ib1,
               acc_sh, sem_i, sem_g, sem_s, sem_z):
    c = lax.axis_index("c")
    s = lax.axis_index("s")
    rows = (rows0, rows1, rows2)
    ibs = (ib0, ib1)
    zero16 = jnp.zeros((16,), jnp.float32)

    def _zrow(r, _):
        def _zcol(j, _):
            rows0[r, pl.ds(j * 16, 16)] = zero16
            return 0
        return lax.fori_loop(0, 8, _zcol, 0)
    lax.fori_loop(0, EB, _zrow, 0)

    def _idx_start(o, q):
        pltpu.async_copy(idx_hbm.at[c, pl.ds(s * TB + o * G3, G3), :, :],
                         ibs[q], sem_i.at[q])

    def _idx_wait(o, q):
        pltpu.make_async_copy(
            idx_hbm.at[c, pl.ds(s * TB + o * G3, G3), :, :],
            ibs[q], sem_i.at[q]).wait()

    def _gather_start(q, i, slot):
        pltpu.async_copy(h_hbm.at[ibs[q].at[i, 0]], rows[slot],
                         sem_g.at[slot])

    def _gather_wait(q, i, slot):
        pltpu.make_async_copy(h_hbm.at[ibs[q].at[i, 0]], rows[slot],
                              sem_g.at[slot]).wait()

    def _scat_start(q, i, slot):
        pltpu.async_copy(rows[slot], acc_sh.at[ibs[q].at[i, 1]],
                         sem_s.at[slot], add=True)

    def _scat_wait(q, i, slot):
        pltpu.make_async_copy(rows[slot], acc_sh.at[ibs[q].at[i, 1]],
                              sem_s.at[slot]).wait()

    _idx_start(jnp.int32(0), 0)

    for k in range(4):
        pltpu.async_copy(rows0, acc_sh.at[pl.ds(s * RT1 + k * 128, 128), :],
                         sem_z)
    pltpu.async_copy(rows0.at[pl.ds(0, RT1 - 512), :],
                     acc_sh.at[pl.ds(s * RT1 + 512, RT1 - 512), :], sem_z)
    for k in range(4):
        pltpu.make_async_copy(rows0, acc_sh.at[pl.ds(s * RT1, 128), :],
                              sem_z).wait()
    pltpu.make_async_copy(rows0.at[pl.ds(0, RT1 - 512), :],
                          acc_sh.at[pl.ds(s * RT1, RT1 - 512), :],
                          sem_z).wait()
    _idx_wait(jnp.int32(0), 0)
    _gather_start(0, 0, 0)
    _gather_start(0, 1, 1)
    plsc.subcore_barrier()

    def _outer(oo, _):
        for h in range(2):
            o = oo * 2 + h
            q = h
            g0 = o * G3

            _gather_wait(q, 0, 0)
            _scat_start(q, 0, 0)
            if h == 0:
                @pl.when(g0 >= 1)
                def _():
                    _scat_wait(1 - q, 2, 2)
            else:
                _scat_wait(1 - q, 2, 2)
            if h == 0:
                _idx_start(o + 1, 1 - q)
            else:
                @pl.when(o + 1 < NG1)
                def _():
                    _idx_start(o + 1, 1 - q)
            _gather_start(q, 2, 2)

            _gather_wait(q, 1, 1)
            _scat_start(q, 1, 1)
            _scat_wait(q, 0, 0)
            if h == 0:
                _idx_wait(o + 1, 1 - q)
                _gather_start(1 - q, 0, 0)
            else:
                @pl.when(o + 1 < NG1)
                def _():
                    _idx_wait(o + 1, 1 - q)
                    _gather_start(1 - q, 0, 0)

            _gather_wait(q, 2, 2)
            _scat_start(q, 2, 2)
            _scat_wait(q, 1, 1)
            if h == 0:
                _gather_start(1 - q, 1, 1)
            else:
                @pl.when(o + 1 < NG1)
                def _():
                    _gather_start(1 - q, 1, 1)
        return 0
    lax.fori_loop(0, NG1 // 2, _outer, 0)
    _scat_wait(1, 2, 2)
    plsc.subcore_barrier()

    pltpu.sync_copy(acc_sh.at[pl.ds(s * RT1, RT1), :],
                    ssum_hbm.at[c, s, :, :])


_agg1 = functools.partial(
    pl.kernel, _agg1_body, mesh=_MESH,
    out_type=jax.ShapeDtypeStruct((2, NT, RT1, 128), jnp.float32),
    scratch_types=[
        pltpu.VMEM((EB, 128), jnp.float32),
        pltpu.VMEM((EB, 128), jnp.float32),
        pltpu.VMEM((EB, 128), jnp.float32),
        pltpu.VMEM((G3, 2, EB), jnp.int32),
        pltpu.VMEM((G3, 2, EB), jnp.int32),
        pltpu.VMEM_SHARED((NPAD1, 128), jnp.float32),
        pltpu.SemaphoreType.DMA((2,)),
        pltpu.SemaphoreType.DMA((3,)),
        pltpu.SemaphoreType.DMA((3,)),
        pltpu.SemaphoreType.DMA,
    ],
)()


def _stage2_body(sa_ref, sb_ref, c0_ref, c1_ref, xr1_ref, wl1a_ref,
                 wl1b_ref, bl1_ref, wp2_ref, bp2_ref, wl2_ref, wr2_ref,
                 s2_ref, r2_ref, cs_ref):
    csum = c0_ref[...] + c1_ref[...]
    cs_ref[...] = csum
    inv = 1.0 / jnp.maximum(csum, 1.0)
    lsum = _dot_t(sa_ref[...], wl1a_ref[...]) + _dot_t(sb_ref[...],
                                                       wl1b_ref[...])
    out1 = jnp.maximum(lsum * inv + bl1_ref[...] + xr1_ref[...], 0.0)
    h2 = jnp.maximum(_dot_t(out1, wp2_ref[...]) + bp2_ref[...], 0.0)
    s2_ref[...] = _dot_t(h2, wl2_ref[...])
    r2_ref[...] = _dot_t(out1, wr2_ref[...])


def _stage2(sa, sb, c0, c1, xr1, Wl1, bl1, Wp2, bp2, Wl2, Wr2):
    return pl.pallas_call(
        _stage2_body,
        grid=(N // ROWS,),
        in_specs=[
            pl.BlockSpec((ROWS, 128), lambda i: (i, 0)),
            pl.BlockSpec((ROWS, 128), lambda i: (i, 0)),
            pl.BlockSpec((ROWS, 1), lambda i: (i, 0)),
            pl.BlockSpec((ROWS, 1), lambda i: (i, 0)),
            pl.BlockSpec((ROWS, H), lambda i: (i, 0)),
            pl.BlockSpec((H, 128), lambda i: (0, 0)),
            pl.BlockSpec((H, 128), lambda i: (0, 0)),
            pl.BlockSpec((1, H), lambda i: (0, 0)),
            pl.BlockSpec((H, H), lambda i: (0, 0)),
            pl.BlockSpec((1, H), lambda i: (0, 0)),
            pl.BlockSpec((1, H), lambda i: (0, 0)),
            pl.BlockSpec((1, H), lambda i: (0, 0)),
        ],
        out_specs=[
            pl.BlockSpec((ROWS, 1), lambda i: (i, 0)),
            pl.BlockSpec((ROWS, 1), lambda i: (i, 0)),
            pl.BlockSpec((ROWS, 1), lambda i: (i, 0)),
        ],
        out_shape=[
            jax.ShapeDtypeStruct((N, 1), jnp.float32),
            jax.ShapeDtypeStruct((N, 1), jnp.float32),
            jax.ShapeDtypeStruct((N, 1), jnp.float32),
        ],
    )(sa, sb, c0.reshape(N, 1), c1.reshape(N, 1), xr1, Wl1[:, :128],
      Wl1[:, 128:], bl1.reshape(1, H), Wp2, bp2.reshape(1, H), Wl2, Wr2)


def _agg2_body(s2_hbm, src_hbm, dst_hbm, cnt_hbm, r2_hbm, out_hbm,
               zcnt, s2_v, src3, dst3, vals0, vals1, vals2, vals3,
               a_v, c_v, r_v, o_v, acc_sh, sem_g, sem_s):
    c = lax.axis_index("c")
    s = lax.axis_index("s")
    vals = (vals0, vals1, vals2, vals3)
    zero16 = jnp.zeros((16,), jnp.float32)

    @pl.when(c == 0)
    def _():
        def _zc(j, _):
            zcnt[pl.ds(j * 16, 16)] = zero16
            return 0
        lax.fori_loop(0, RT2 // 16, _zc, 0)
        pltpu.async_copy(src_hbm.at[s, :, :], src3, sem_g.at[0])
        pltpu.async_copy(dst_hbm.at[s, :, :], dst3, sem_g.at[1])
        pltpu.async_copy(s2_hbm, s2_v, sem_g.at[2])
        pltpu.sync_copy(zcnt, acc_sh.at[pl.ds(s * RT2, RT2)])
        pltpu.make_async_copy(src_hbm.at[s, :, :], src3,
                              sem_g.at[0]).wait()
        pltpu.make_async_copy(dst_hbm.at[s, :, :], dst3,
                              sem_g.at[1]).wait()
        pltpu.make_async_copy(s2_hbm, s2_v, sem_g.at[2]).wait()
    plsc.subcore_barrier()

    @pl.when(c == 0)
    def _():
        def _scat_wait(g, slot):
            pltpu.make_async_copy(vals[slot], acc_sh.at[dst3.at[g]],
                                  sem_s.at[slot]).wait()

        def _group(o, _):
            for i in range(NB2):
                g = o * NB2 + i

                @pl.when(g >= NB2)
                def _():
                    _scat_wait(g, i)

                def _gather(j, _):
                    idx16 = src3[g, pl.ds(j * 16, 16)]
                    row16 = lax.shift_right_logical(idx16, 7)
                    col16 = lax.bitwise_and(idx16, 127)
                    vals[i][pl.ds(j * 16, 16)] = plsc.load_gather(
                        s2_v, [row16, col16])
                    return 0
                lax.fori_loop(0, EB // 16, _gather, 0)
                pltpu.async_copy(vals[i], acc_sh.at[dst3.at[g]],
                                 sem_s.at[i], add=True)
            return 0
        lax.fori_loop(0, TB // NB2, _group, 0)
        for i in range(NB2):
            _scat_wait(jnp.int32(TB - NB2 + i), i)
    plsc.subcore_barrier()

    @pl.when(c == 0)
    def _():
        pltpu.sync_copy(acc_sh.at[pl.ds(s * RT2, RT2)], a_v)
        pltpu.sync_copy(cnt_hbm.at[pl.ds(s * RT2, RT2)], c_v)
        pltpu.sync_copy(r2_hbm.at[pl.ds(s * RT2, RT2)], r_v)

        def _fin(j, _):
            sl = pl.ds(j * 16, 16)
            z = a_v[sl] / jnp.maximum(c_v[sl], 1.0) + r_v[sl]
            o_v[sl] = 1.0 / (1.0 + jnp.exp(-z))
            return 0
        lax.fori_loop(0, RT2 // 16, _fin, 0)
        pltpu.sync_copy(o_v, out_hbm.at[pl.ds(s * RT2, RT2)])


_agg2 = functools.partial(
    pl.kernel, _agg2_body, mesh=_MESH,
    compiler_params=pltpu.CompilerParams(needs_layout_passes=False),
    out_type=jax.ShapeDtypeStruct((NPAD2,), jnp.float32),
    scratch_types=[
        pltpu.VMEM((RT2,), jnp.float32),
        pltpu.VMEM((NPAD2 // 128, 128), jnp.float32),
        pltpu.VMEM((TB, EB), jnp.int32),
        pltpu.VMEM((TB, EB), jnp.int32),
        pltpu.VMEM((EB,), jnp.float32),
        pltpu.VMEM((EB,), jnp.float32),
        pltpu.VMEM((EB,), jnp.float32),
        pltpu.VMEM((EB,), jnp.float32),
        pltpu.VMEM((RT2,), jnp.float32),
        pltpu.VMEM((RT2,), jnp.float32),
        pltpu.VMEM((RT2,), jnp.float32),
        pltpu.VMEM((RT2,), jnp.float32),
        pltpu.VMEM_SHARED((NPAD2,), jnp.float32),
        pltpu.SemaphoreType.DMA((NB2,)),
        pltpu.SemaphoreType.DMA((NB2,)),
    ],
)()


def kernel(x, edge_index, Wp1, bp1, Wl1, bl1, Wr1, Wp2, bp2, Wl2, bl2, Wr2):
    src = edge_index[0]
    dst = edge_index[1]
    pad = EPAD - E
    src_p = jnp.concatenate([src, jnp.zeros((pad,), jnp.int32)])
    dst_p = jnp.concatenate([dst, jnp.full((pad,), N, jnp.int32)])
    src2 = src_p.reshape(EROWS, EB)
    dst2 = dst_p.reshape(EROWS, EB)
    idxcat = jnp.stack([
        jnp.stack([src2 * 2, dst2], axis=1),
        jnp.stack([src2 * 2 + 1, dst2], axis=1),
    ])

    cnt2 = _deg(dst2.reshape(2 * NT, TB // 2, EB))
    h1, xr1 = _stage1(x, Wp1, bp1, Wr1)
    ssum = _agg1(h1.reshape(2 * N, 128), idxcat).reshape(2, NPAD1, 128)
    s2, r2, csum = _stage2(ssum[0, :N, :], ssum[1, :N, :], cnt2[0, :N],
                           cnt2[1, :N], xr1, Wl1, bl1, Wp2, bp2, Wl2, Wr2)
    zpad = jnp.zeros((NPAD2 - N,), jnp.float32)
    s2_p = jnp.concatenate([s2.reshape(N), zpad])
    r2_p = jnp.concatenate([r2.reshape(N) + bl2[0], zpad])
    cs_p = jnp.concatenate([csum.reshape(N), zpad])
    src3d = src2.reshape(NT, TB, EB)
    dst3d = dst2.reshape(NT, TB, EB)
    out = _agg2(s2_p.reshape(NPAD2 // 128, 128), src3d, dst3d, cs_p, r2_p)
    return out[:N].reshape(N, 1)


# speedup vs baseline: 1.0211x; 1.0052x over previous
import functools

import jax
import jax.numpy as jnp
from jax import lax
from jax.experimental import pallas as pl
from jax.experimental.pallas import tpu as pltpu
from jax.experimental.pallas import tpu_sc as plsc

N = 10000
E = 160000
D = 256
H = 512

ROWS = 1000

EB = 128
NT = 16
TB = 84
EPAD = NT * TB * EB
EROWS = EPAD // EB
NPAD1 = 10016
RT1 = NPAD1 // NT
NPAD2 = 10240
RT2 = NPAD2 // NT
NB2 = 4

_MESH = plsc.VectorSubcoreMesh(core_axis_name="c", subcore_axis_name="s")


def _dot_t(a, b):
    return lax.dot_general(a, b, (((1,), (1,)), ((), ())),
                           preferred_element_type=jnp.float32)


def _stage1_body(x_ref, wp1_ref, bp1_ref, wr1_ref, h1_ref, xr1_ref):
    xb = x_ref[...]
    h1_ref[...] = jnp.maximum(_dot_t(xb, wp1_ref[...]) + bp1_ref[...], 0.0)
    xr1_ref[...] = _dot_t(xb, wr1_ref[...])


def _stage1(x, Wp1, bp1, Wr1):
    return pl.pallas_call(
        _stage1_body,
        grid=(N // ROWS,),
        in_specs=[
            pl.BlockSpec((ROWS, D), lambda i: (i, 0)),
            pl.BlockSpec((D, D), lambda i: (0, 0)),
            pl.BlockSpec((1, D), lambda i: (0, 0)),
            pl.BlockSpec((H, D), lambda i: (0, 0)),
        ],
        out_specs=[
            pl.BlockSpec((ROWS, D), lambda i: (i, 0)),
            pl.BlockSpec((ROWS, H), lambda i: (i, 0)),
        ],
        out_shape=[
            jax.ShapeDtypeStruct((N, D), jnp.float32),
            jax.ShapeDtypeStruct((N, H), jnp.float32),
        ],
    )(x, Wp1, bp1.reshape(1, D), Wr1)


def _deg_body(dst_hbm, cnt_hbm, ones_v, zc, dst3, cnt_sh, sem_c, sem_z):
    c = lax.axis_index("c")
    s = lax.axis_index("s")
    zero16 = jnp.zeros((16,), jnp.float32)
    half = TB // 2
    w = c * NT + s

    def _ones(j, _):
        ones_v[pl.ds(j * 16, 16)] = zero16 + 1.0
        return 0
    lax.fori_loop(0, EB // 16, _ones, 0)

    def _zc(j, _):
        zc[pl.ds(j * 16, 16)] = zero16
        return 0
    lax.fori_loop(0, RT2 // 16, _zc, 0)

    pltpu.async_copy(dst_hbm.at[w, :, :], dst3, sem_z)
    pltpu.sync_copy(zc, cnt_sh.at[pl.ds(s * RT2, RT2)])
    pltpu.make_async_copy(dst_hbm.at[w, :, :], dst3, sem_z).wait()
    plsc.subcore_barrier()

    def _wait_scat(g, slot):
        pltpu.make_async_copy(ones_v, cnt_sh.at[dst3.at[g]],
                              sem_c.at[slot]).wait()

    def _outer(oo, _):
        for j in range(3):
            g = oo * 3 + j

            def _lag():
                _wait_scat(g - 2, (j + 1) % 3)
            if j < 2:
                @pl.when(g >= 2)
                def _():
                    _lag()
            else:
                _lag()
            pltpu.async_copy(ones_v, cnt_sh.at[dst3.at[g]],
                             sem_c.at[j % 3], add=True)
        return 0
    lax.fori_loop(0, half // 3, _outer, 0)
    _wait_scat(jnp.int32(half - 2), (half - 2) % 3)
    _wait_scat(jnp.int32(half - 1), (half - 1) % 3)
    plsc.subcore_barrier()
    pltpu.sync_copy(cnt_sh.at[pl.ds(s * RT2, RT2)],
                    cnt_hbm.at[c, pl.ds(s * RT2, RT2)])


_deg = functools.partial(
    pl.kernel, _deg_body, mesh=_MESH,
    out_type=jax.ShapeDtypeStruct((2, NPAD2), jnp.float32),
    scratch_types=[
        pltpu.VMEM((EB,), jnp.float32),
        pltpu.VMEM((RT2,), jnp.float32),
        pltpu.VMEM((TB // 2, EB), jnp.int32),
        pltpu.VMEM_SHARED((NPAD2,), jnp.float32),
        pltpu.SemaphoreType.DMA((3,)),
        pltpu.SemaphoreType.DMA,
    ],
)()


G3 = 21
NG1 = TB // G3


def _agg1_body(h_hbm, idx_hbm, ssum_hbm,
               rows0, rows1, ib0, ib1,
               acc_sh, sem_i, sem_g, sem_s, sem_z):
    c = lax.axis_index("c")
    s = lax.axis_index("s")
    rows = (rows0, rows1)
    ibs = (ib0, ib1)
    zero16 = jnp.zeros((16,), jnp.float32)

    def _zrow(r, _):
        def _zcol(j, _):
            rows0[r, pl.ds(j * 16, 16)] = zero16
            return 0
        return lax.fori_loop(0, 8, _zcol, 0)
    lax.fori_loop(0, EB, _zrow, 0)

    def _idx_start(o, q):
        pltpu.async_copy(idx_hbm.at[c, pl.ds(s * TB + o * G3, G3), :, :],
                         ibs[q], sem_i.at[q])

    def _idx_wait(o, q):
        pltpu.make_async_copy(
            idx_hbm.at[c, pl.ds(s * TB + o * G3, G3), :, :],
            ibs[q], sem_i.at[q]).wait()

    def _gather_start(q, i, slot):
        pltpu.async_copy(h_hbm.at[ibs[q].at[i, 0]], rows[slot],
                         sem_g.at[slot])

    def _gather_wait(q, i, slot):
        pltpu.make_async_copy(h_hbm.at[ibs[q].at[i, 0]], rows[slot],
                              sem_g.at[slot]).wait()

    def _scat_start(q, i, slot):
        pltpu.async_copy(rows[slot], acc_sh.at[ibs[q].at[i, 1]],
                         sem_s.at[slot], add=True)

    def _scat_wait(q, i, slot):
        pltpu.make_async_copy(rows[slot], acc_sh.at[ibs[q].at[i, 1]],
                              sem_s.at[slot]).wait()

    _idx_start(jnp.int32(0), 0)

    for k in range(4):
        pltpu.async_copy(rows0, acc_sh.at[pl.ds(s * RT1 + k * 128, 128), :],
                         sem_z)
    pltpu.async_copy(rows0.at[pl.ds(0, RT1 - 512), :],
                     acc_sh.at[pl.ds(s * RT1 + 512, RT1 - 512), :], sem_z)
    for k in range(4):
        pltpu.make_async_copy(rows0, acc_sh.at[pl.ds(s * RT1, 128), :],
                              sem_z).wait()
    pltpu.make_async_copy(rows0.at[pl.ds(0, RT1 - 512), :],
                          acc_sh.at[pl.ds(s * RT1, RT1 - 512), :],
                          sem_z).wait()
    _idx_wait(jnp.int32(0), 0)
    _gather_start(0, 0, 0)
    _gather_start(0, 1, 1)
    plsc.subcore_barrier()

    def _outer(oo, _):
        for h in range(2):
            o = oo * 2 + h
            q = h

            @pl.when(o + 1 < NG1)
            def _():
                _idx_start(o + 1, 1 - q)
            for i in range(G3):
                sl = (h + i) % 2
                _gather_wait(q, i, sl)
                _scat_start(q, i, sl)
                _scat_wait(q, i, sl)
                if i < G3 - 2:
                    _gather_start(q, i + 2, sl)
                elif i == G3 - 2:
                    @pl.when(o + 1 < NG1)
                    def _():
                        _idx_wait(o + 1, 1 - q)
                        _gather_start(1 - q, 0, (h + 1) % 2)
                else:
                    @pl.when(o + 1 < NG1)
                    def _():
                        _gather_start(1 - q, 1, h % 2)
        return 0
    lax.fori_loop(0, NG1 // 2, _outer, 0)
    plsc.subcore_barrier()

    pltpu.sync_copy(acc_sh.at[pl.ds(s * RT1, RT1), :],
                    ssum_hbm.at[c, s, :, :])


_agg1 = functools.partial(
    pl.kernel, _agg1_body, mesh=_MESH,
    out_type=jax.ShapeDtypeStruct((2, NT, RT1, 128), jnp.float32),
    scratch_types=[
        pltpu.VMEM((EB, 128), jnp.float32),
        pltpu.VMEM((EB, 128), jnp.float32),
        pltpu.VMEM((G3, 2, EB), jnp.int32),
        pltpu.VMEM((G3, 2, EB), jnp.int32),
        pltpu.VMEM_SHARED((NPAD1, 128), jnp.float32),
        pltpu.SemaphoreType.DMA((2,)),
        pltpu.SemaphoreType.DMA((2,)),
        pltpu.SemaphoreType.DMA((2,)),
        pltpu.SemaphoreType.DMA,
    ],
)()


def _stage2_body(sa_ref, sb_ref, c0_ref, c1_ref, xr1_ref, wl1a_ref,
                 wl1b_ref, bl1_ref, wp2_ref, bp2_ref, wl2_ref, wr2_ref,
                 s2_ref, r2_ref, cs_ref):
    csum = c0_ref[...] + c1_ref[...]
    cs_ref[...] = csum
    inv = 1.0 / jnp.maximum(csum, 1.0)
    lsum = _dot_t(sa_ref[...], wl1a_ref[...]) + _dot_t(sb_ref[...],
                                                       wl1b_ref[...])
    out1 = jnp.maximum(lsum * inv + bl1_ref[...] + xr1_ref[...], 0.0)
    h2 = jnp.maximum(_dot_t(out1, wp2_ref[...]) + bp2_ref[...], 0.0)
    s2_ref[...] = _dot_t(h2, wl2_ref[...])
    r2_ref[...] = _dot_t(out1, wr2_ref[...])


def _stage2(sa, sb, c0, c1, xr1, Wl1, bl1, Wp2, bp2, Wl2, Wr2):
    return pl.pallas_call(
        _stage2_body,
        grid=(N // ROWS,),
        in_specs=[
            pl.BlockSpec((ROWS, 128), lambda i: (i, 0)),
            pl.BlockSpec((ROWS, 128), lambda i: (i, 0)),
            pl.BlockSpec((ROWS, 1), lambda i: (i, 0)),
            pl.BlockSpec((ROWS, 1), lambda i: (i, 0)),
            pl.BlockSpec((ROWS, H), lambda i: (i, 0)),
            pl.BlockSpec((H, 128), lambda i: (0, 0)),
            pl.BlockSpec((H, 128), lambda i: (0, 0)),
            pl.BlockSpec((1, H), lambda i: (0, 0)),
            pl.BlockSpec((H, H), lambda i: (0, 0)),
            pl.BlockSpec((1, H), lambda i: (0, 0)),
            pl.BlockSpec((1, H), lambda i: (0, 0)),
            pl.BlockSpec((1, H), lambda i: (0, 0)),
        ],
        out_specs=[
            pl.BlockSpec((ROWS, 1), lambda i: (i, 0)),
            pl.BlockSpec((ROWS, 1), lambda i: (i, 0)),
            pl.BlockSpec((ROWS, 1), lambda i: (i, 0)),
        ],
        out_shape=[
            jax.ShapeDtypeStruct((N, 1), jnp.float32),
            jax.ShapeDtypeStruct((N, 1), jnp.float32),
            jax.ShapeDtypeStruct((N, 1), jnp.float32),
        ],
    )(sa, sb, c0.reshape(N, 1), c1.reshape(N, 1), xr1, Wl1[:, :128],
      Wl1[:, 128:], bl1.reshape(1, H), Wp2, bp2.reshape(1, H), Wl2, Wr2)


def _agg2_body(s2_hbm, src_hbm, dst_hbm, cnt_hbm, r2_hbm, out_hbm,
               zcnt, s2_v, src3, dst3, vals0, vals1, vals2, vals3,
               a_v, c_v, r_v, o_v, acc_sh, sem_g, sem_s):
    c = lax.axis_index("c")
    s = lax.axis_index("s")
    vals = (vals0, vals1, vals2, vals3)
    zero16 = jnp.zeros((16,), jnp.float32)

    @pl.when(c == 0)
    def _():
        def _zc(j, _):
            zcnt[pl.ds(j * 16, 16)] = zero16
            return 0
        lax.fori_loop(0, RT2 // 16, _zc, 0)
        pltpu.async_copy(src_hbm.at[s, :, :], src3, sem_g.at[0])
        pltpu.async_copy(dst_hbm.at[s, :, :], dst3, sem_g.at[1])
        pltpu.async_copy(s2_hbm, s2_v, sem_g.at[2])
        pltpu.sync_copy(zcnt, acc_sh.at[pl.ds(s * RT2, RT2)])
        pltpu.make_async_copy(src_hbm.at[s, :, :], src3,
                              sem_g.at[0]).wait()
        pltpu.make_async_copy(dst_hbm.at[s, :, :], dst3,
                              sem_g.at[1]).wait()
        pltpu.make_async_copy(s2_hbm, s2_v, sem_g.at[2]).wait()
    plsc.subcore_barrier()

    @pl.when(c == 0)
    def _():
        def _scat_wait(g, slot):
            pltpu.make_async_copy(vals[slot], acc_sh.at[dst3.at[g]],
                                  sem_s.at[slot]).wait()

        def _group(o, _):
            for i in range(NB2):
                g = o * NB2 + i

                @pl.when(g >= NB2)
                def _():
                    _scat_wait(g, i)

                def _gather(j, _):
                    idx16 = src3[g, pl.ds(j * 16, 16)]
                    row16 = lax.shift_right_logical(idx16, 7)
                    col16 = lax.bitwise_and(idx16, 127)
                    vals[i][pl.ds(j * 16, 16)] = plsc.load_gather(
                        s2_v, [row16, col16])
                    return 0
                lax.fori_loop(0, EB // 16, _gather, 0)
                pltpu.async_copy(vals[i], acc_sh.at[dst3.at[g]],
                                 sem_s.at[i], add=True)
            return 0
        lax.fori_loop(0, TB // NB2, _group, 0)
        for i in range(NB2):
            _scat_wait(jnp.int32(TB - NB2 + i), i)
    plsc.subcore_barrier()

    @pl.when(c == 0)
    def _():
        pltpu.sync_copy(acc_sh.at[pl.ds(s * RT2, RT2)], a_v)
        pltpu.sync_copy(cnt_hbm.at[pl.ds(s * RT2, RT2)], c_v)
        pltpu.sync_copy(r2_hbm.at[pl.ds(s * RT2, RT2)], r_v)

        def _fin(j, _):
            sl = pl.ds(j * 16, 16)
            z = a_v[sl] / jnp.maximum(c_v[sl], 1.0) + r_v[sl]
            o_v[sl] = 1.0 / (1.0 + jnp.exp(-z))
            return 0
        lax.fori_loop(0, RT2 // 16, _fin, 0)
        pltpu.sync_copy(o_v, out_hbm.at[pl.ds(s * RT2, RT2)])


_agg2 = functools.partial(
    pl.kernel, _agg2_body, mesh=_MESH,
    compiler_params=pltpu.CompilerParams(needs_layout_passes=False),
    out_type=jax.ShapeDtypeStruct((NPAD2,), jnp.float32),
    scratch_types=[
        pltpu.VMEM((RT2,), jnp.float32),
        pltpu.VMEM((NPAD2 // 128, 128), jnp.float32),
        pltpu.VMEM((TB, EB), jnp.int32),
        pltpu.VMEM((TB, EB), jnp.int32),
        pltpu.VMEM((EB,), jnp.float32),
        pltpu.VMEM((EB,), jnp.float32),
        pltpu.VMEM((EB,), jnp.float32),
        pltpu.VMEM((EB,), jnp.float32),
        pltpu.VMEM((RT2,), jnp.float32),
        pltpu.VMEM((RT2,), jnp.float32),
        pltpu.VMEM((RT2,), jnp.float32),
        pltpu.VMEM((RT2,), jnp.float32),
        pltpu.VMEM_SHARED((NPAD2,), jnp.float32),
        pltpu.SemaphoreType.DMA((NB2,)),
        pltpu.SemaphoreType.DMA((NB2,)),
    ],
)()


def kernel(x, edge_index, Wp1, bp1, Wl1, bl1, Wr1, Wp2, bp2, Wl2, bl2, Wr2):
    src = edge_index[0]
    dst = edge_index[1]
    pad = EPAD - E
    src_p = jnp.concatenate([src, jnp.zeros((pad,), jnp.int32)])
    dst_p = jnp.concatenate([dst, jnp.full((pad,), N, jnp.int32)])
    src2 = src_p.reshape(EROWS, EB)
    dst2 = dst_p.reshape(EROWS, EB)
    idxcat = jnp.stack([
        jnp.stack([src2 * 2, dst2], axis=1),
        jnp.stack([src2 * 2 + 1, dst2], axis=1),
    ])

    cnt2 = _deg(dst2.reshape(2 * NT, TB // 2, EB))
    h1, xr1 = _stage1(x, Wp1, bp1, Wr1)
    ssum = _agg1(h1.reshape(2 * N, 128), idxcat).reshape(2, NPAD1, 128)
    s2, r2, csum = _stage2(ssum[0, :N, :], ssum[1, :N, :], cnt2[0, :N],
                           cnt2[1, :N], xr1, Wl1, bl1, Wp2, bp2, Wl2, Wr2)
    zpad = jnp.zeros((NPAD2 - N,), jnp.float32)
    s2_p = jnp.concatenate([s2.reshape(N), zpad])
    r2_p = jnp.concatenate([r2.reshape(N) + bl2[0], zpad])
    cs_p = jnp.concatenate([csum.reshape(N), zpad])
    src3d = src2.reshape(NT, TB, EB)
    dst3d = dst2.reshape(NT, TB, EB)
    out = _agg2(s2_p.reshape(NPAD2 // 128, 128), src3d, dst3d, cs_p, r2_p)
    return out[:N].reshape(N, 1)


# speedup vs baseline: 1.9705x; 1.9298x over previous
import functools

import jax
import jax.numpy as jnp
from jax import lax
from jax.experimental import pallas as pl
from jax.experimental.pallas import tpu as pltpu
from jax.experimental.pallas import tpu_sc as plsc

N = 10000
E = 160000
D = 256
H = 512

ROWS = 1000

EB = 128
NT = 16
EPAD = 163840
TILE_E = EPAD // NT
TILE_B = TILE_E // EB
NPAD = 10240
ROWS_T = NPAD // NT
NBUF = 5
G = 10
NGRP = TILE_B // G

_MESH = plsc.VectorSubcoreMesh(core_axis_name="c", subcore_axis_name="s")


def _dot_t(a, b):
    return lax.dot_general(a, b, (((1,), (1,)), ((), ())),
                           preferred_element_type=jnp.float32)


def _stage1_body(x_ref, wp1_ref, bp1_ref, wr1_ref, h1_ref, xr1_ref):
    xb = x_ref[...]
    h1_ref[...] = jnp.maximum(_dot_t(xb, wp1_ref[...]) + bp1_ref[...], 0.0)
    xr1_ref[...] = _dot_t(xb, wr1_ref[...])


def _stage1(x, Wp1, bp1, Wr1):
    return pl.pallas_call(
        _stage1_body,
        grid=(N // ROWS,),
        in_specs=[
            pl.BlockSpec((ROWS, D), lambda i: (i, 0)),
            pl.BlockSpec((D, D), lambda i: (0, 0)),
            pl.BlockSpec((1, D), lambda i: (0, 0)),
            pl.BlockSpec((H, D), lambda i: (0, 0)),
        ],
        out_specs=[
            pl.BlockSpec((ROWS, D), lambda i: (i, 0)),
            pl.BlockSpec((ROWS, H), lambda i: (i, 0)),
        ],
        out_shape=[
            jax.ShapeDtypeStruct((N, D), jnp.float32),
            jax.ShapeDtypeStruct((N, H), jnp.float32),
        ],
    )(x, Wp1, bp1.reshape(1, D), Wr1)


def _agg1_body(h_hbm, idx_hbm, ssum_hbm, cnt_hbm,
               zb, ones_v, ib0, ib1, rows0, rows1,
               acc_sh, cnt_sh, sem_i, sem_g, sem_s, sem_c, sem_z):
    c = lax.axis_index("c")
    s = lax.axis_index("s")
    rows = (rows0, rows1)
    ibs = (ib0, ib1)
    zero16 = jnp.zeros((16,), jnp.float32)

    def _zrow(r, _):
        def _zcol(j, _):
            zb[r, pl.ds(j * 16, 16)] = zero16
            return 0
        return lax.fori_loop(0, 8, _zcol, 0)
    lax.fori_loop(0, 32, _zrow, 0)

    def _ones(j, _):
        ones_v[pl.ds(j * 16, 16)] = zero16 + 1.0
        return 0
    lax.fori_loop(0, EB // 16, _ones, 0)

    def _idx_start(o, q):
        pltpu.async_copy(idx_hbm.at[c, pl.ds(s * TILE_B + o * G, G), :, :],
                         ibs[q], sem_i.at[q])

    def _idx_wait(o, q):
        pltpu.make_async_copy(
            idx_hbm.at[c, pl.ds(s * TILE_B + o * G, G), :, :],
            ibs[q], sem_i.at[q]).wait()

    def _gather_start(q, i, slot):
        pltpu.async_copy(h_hbm.at[ibs[q].at[i, 0]], rows[slot],
                         sem_g.at[slot])

    def _gather_wait(q, i, slot):
        pltpu.make_async_copy(h_hbm.at[ibs[q].at[i, 0]], rows[slot],
                              sem_g.at[slot]).wait()

    _idx_start(jnp.int32(0), 0)

    for i in range(ROWS_T // 32):
        pltpu.async_copy(zb, acc_sh.at[pl.ds(s * ROWS_T + i * 32, 32), :],
                         sem_z)
    for i in range(ROWS_T // 128):
        pltpu.async_copy(zb.at[0, :],
                         cnt_sh.at[pl.ds(s * ROWS_T + i * 128, 128)], sem_z)
    for i in range(ROWS_T // 32):
        pltpu.make_async_copy(zb, acc_sh.at[pl.ds(s * ROWS_T, 32), :],
                              sem_z).wait()
    for i in range(ROWS_T // 128):
        pltpu.make_async_copy(zb.at[0, :], cnt_sh.at[pl.ds(s * ROWS_T, 128)],
                              sem_z).wait()
    _idx_wait(jnp.int32(0), 0)
    _idx_start(jnp.int32(1), 1)
    _gather_start(0, 0, 0)
    _gather_start(0, 1, 1)
    plsc.subcore_barrier()

    def _outer(oo, _):
        for q in range(2):
            o = oo * 2 + q
            g0 = o * G
            for i in range(G):
                si = i % 2
                g = g0 + i
                _gather_wait(q, i, si)
                pltpu.async_copy(rows[si], acc_sh.at[ibs[q].at[i, 1]],
                                 sem_s.at[si], add=True)

                @pl.when(c == 0)
                def _():
                    @pl.when(g >= 4)
                    def _():
                        pltpu.make_async_copy(
                            ones_v, cnt_sh.at[ibs[q].at[i, 1]],
                            sem_c).wait()
                    pltpu.async_copy(ones_v, cnt_sh.at[ibs[q].at[i, 1]],
                                     sem_c, add=True)

                pltpu.make_async_copy(rows[si], acc_sh.at[ibs[q].at[i, 1]],
                                      sem_s.at[si]).wait()
                if i < G - 2:
                    _gather_start(q, i + 2, si)

            @pl.when(o < NGRP - 1)
            def _():
                _idx_wait(o + 1, 1 - q)
                _gather_start(1 - q, 0, 0)
                _gather_start(1 - q, 1, 1)

                @pl.when(o < NGRP - 2)
                def _():
                    _idx_start(o + 2, q)
        return 0
    lax.fori_loop(0, NGRP // 2, _outer, 0)

    @pl.when(c == 0)
    def _():
        for i in range(4):
            pltpu.make_async_copy(ones_v,
                                  cnt_sh.at[ibs[0].at[jnp.int32(i), 1]],
                                  sem_c).wait()
    plsc.subcore_barrier()

    pltpu.sync_copy(acc_sh.at[pl.ds(s * ROWS_T, ROWS_T), :],
                    ssum_hbm.at[c, pl.ds(s * ROWS_T, ROWS_T), :])

    @pl.when(c == 0)
    def _():
        pltpu.sync_copy(cnt_sh.at[pl.ds(s * ROWS_T, ROWS_T)],
                        cnt_hbm.at[pl.ds(s * ROWS_T, ROWS_T)])


_agg1 = functools.partial(
    pl.kernel, _agg1_body, mesh=_MESH,
    out_type=[
        jax.ShapeDtypeStruct((2, NPAD, 128), jnp.float32),
        jax.ShapeDtypeStruct((NPAD,), jnp.float32),
    ],
    scratch_types=[
        pltpu.VMEM((32, 128), jnp.float32),
        pltpu.VMEM((EB,), jnp.float32),
        pltpu.VMEM((G, 2, EB), jnp.int32),
        pltpu.VMEM((G, 2, EB), jnp.int32),
        pltpu.VMEM((EB, 128), jnp.float32),
        pltpu.VMEM((EB, 128), jnp.float32),
        pltpu.VMEM_SHARED((NPAD, 128), jnp.float32),
        pltpu.VMEM_SHARED((NPAD,), jnp.float32),
        pltpu.SemaphoreType.DMA((2,)),
        pltpu.SemaphoreType.DMA((2,)),
        pltpu.SemaphoreType.DMA((2,)),
        pltpu.SemaphoreType.DMA,
        pltpu.SemaphoreType.DMA,
    ],
)()


def _stage2_body(sa_ref, sb_ref, cnt_ref, xr1_ref, wl1a_ref, wl1b_ref,
                 bl1_ref, wp2_ref, bp2_ref, wl2_ref, wr2_ref,
                 s2_ref, r2_ref):
    inv = 1.0 / jnp.maximum(cnt_ref[...], 1.0)
    lsum = _dot_t(sa_ref[...], wl1a_ref[...]) + _dot_t(sb_ref[...],
                                                       wl1b_ref[...])
    out1 = jnp.maximum(lsum * inv + bl1_ref[...] + xr1_ref[...], 0.0)
    h2 = jnp.maximum(_dot_t(out1, wp2_ref[...]) + bp2_ref[...], 0.0)
    s2_ref[...] = _dot_t(h2, wl2_ref[...])
    r2_ref[...] = _dot_t(out1, wr2_ref[...])


def _stage2(sa, sb, cnt, xr1, Wl1, bl1, Wp2, bp2, Wl2, Wr2):
    return pl.pallas_call(
        _stage2_body,
        grid=(N // ROWS,),
        in_specs=[
            pl.BlockSpec((ROWS, 128), lambda i: (i, 0)),
            pl.BlockSpec((ROWS, 128), lambda i: (i, 0)),
            pl.BlockSpec((ROWS, 1), lambda i: (i, 0)),
            pl.BlockSpec((ROWS, H), lambda i: (i, 0)),
            pl.BlockSpec((H, 128), lambda i: (0, 0)),
            pl.BlockSpec((H, 128), lambda i: (0, 0)),
            pl.BlockSpec((1, H), lambda i: (0, 0)),
            pl.BlockSpec((H, H), lambda i: (0, 0)),
            pl.BlockSpec((1, H), lambda i: (0, 0)),
            pl.BlockSpec((1, H), lambda i: (0, 0)),
            pl.BlockSpec((1, H), lambda i: (0, 0)),
        ],
        out_specs=[
            pl.BlockSpec((ROWS, 1), lambda i: (i, 0)),
            pl.BlockSpec((ROWS, 1), lambda i: (i, 0)),
        ],
        out_shape=[
            jax.ShapeDtypeStruct((N, 1), jnp.float32),
            jax.ShapeDtypeStruct((N, 1), jnp.float32),
        ],
    )(sa, sb, cnt.reshape(N, 1), xr1, Wl1[:, :128], Wl1[:, 128:],
      bl1.reshape(1, H), Wp2, bp2.reshape(1, H), Wl2, Wr2)


def _agg2_body(s2_hbm, src_hbm, dst_hbm, cnt_hbm, r2_hbm, out_hbm,
               zcnt, s2_v, src3, dst3, vals0, vals1, vals2, vals3, vals4,
               a_v, c_v, r_v, o_v, acc_sh, sem_g, sem_s):
    c = lax.axis_index("c")
    s = lax.axis_index("s")
    vals = (vals0, vals1, vals2, vals3, vals4)
    zero16 = jnp.zeros((16,), jnp.float32)

    @pl.when(c == 0)
    def _():
        def _zc(j, _):
            zcnt[pl.ds(j * 16, 16)] = zero16
            return 0
        lax.fori_loop(0, ROWS_T // 16, _zc, 0)
        pltpu.async_copy(src_hbm.at[pl.ds(s * TILE_B, TILE_B), :], src3,
                         sem_g.at[0])
        pltpu.async_copy(dst_hbm.at[pl.ds(s * TILE_B, TILE_B), :], dst3,
                         sem_g.at[1])
        pltpu.async_copy(s2_hbm, s2_v, sem_g.at[2])
        pltpu.sync_copy(zcnt, acc_sh.at[pl.ds(s * ROWS_T, ROWS_T)])
        pltpu.make_async_copy(src_hbm.at[pl.ds(s * TILE_B, TILE_B), :],
                              src3, sem_g.at[0]).wait()
        pltpu.make_async_copy(dst_hbm.at[pl.ds(s * TILE_B, TILE_B), :],
                              dst3, sem_g.at[1]).wait()
        pltpu.make_async_copy(s2_hbm, s2_v, sem_g.at[2]).wait()
    plsc.subcore_barrier()

    @pl.when(c == 0)
    def _():
        def _scat_wait(g, slot):
            pltpu.make_async_copy(vals[slot], acc_sh.at[dst3.at[g]],
                                  sem_s.at[slot]).wait()

        def _group(o, _):
            for i in range(NBUF):
                g = o * NBUF + i

                @pl.when(g >= NBUF)
                def _():
                    _scat_wait(g, i)

                def _gather(j, _):
                    idx16 = src3[g, pl.ds(j * 16, 16)]
                    row16 = lax.shift_right_logical(idx16, 7)
                    col16 = lax.bitwise_and(idx16, 127)
                    vals[i][pl.ds(j * 16, 16)] = plsc.load_gather(
                        s2_v, [row16, col16])
                    return 0
                lax.fori_loop(0, EB // 16, _gather, 0)
                pltpu.async_copy(vals[i], acc_sh.at[dst3.at[g]],
                                 sem_s.at[i], add=True)
            return 0
        lax.fori_loop(0, TILE_B // NBUF, _group, 0)
        for i in range(NBUF):
            _scat_wait(jnp.int32(TILE_B - NBUF + i), i)
    plsc.subcore_barrier()

    @pl.when(c == 0)
    def _():
        pltpu.sync_copy(acc_sh.at[pl.ds(s * ROWS_T, ROWS_T)], a_v)
        pltpu.sync_copy(cnt_hbm.at[pl.ds(s * ROWS_T, ROWS_T)], c_v)
        pltpu.sync_copy(r2_hbm.at[pl.ds(s * ROWS_T, ROWS_T)], r_v)

        def _fin(j, _):
            sl = pl.ds(j * 16, 16)
            z = a_v[sl] / jnp.maximum(c_v[sl], 1.0) + r_v[sl]
            o_v[sl] = 1.0 / (1.0 + jnp.exp(-z))
            return 0
        lax.fori_loop(0, ROWS_T // 16, _fin, 0)
        pltpu.sync_copy(o_v, out_hbm.at[pl.ds(s * ROWS_T, ROWS_T)])


_agg2 = functools.partial(
    pl.kernel, _agg2_body, mesh=_MESH,
    compiler_params=pltpu.CompilerParams(needs_layout_passes=False),
    out_type=jax.ShapeDtypeStruct((NPAD,), jnp.float32),
    scratch_types=[
        pltpu.VMEM((ROWS_T,), jnp.float32),
        pltpu.VMEM((NPAD // 128, 128), jnp.float32),
        pltpu.VMEM((TILE_B, EB), jnp.int32),
        pltpu.VMEM((TILE_B, EB), jnp.int32),
        pltpu.VMEM((EB,), jnp.float32),
        pltpu.VMEM((EB,), jnp.float32),
        pltpu.VMEM((EB,), jnp.float32),
        pltpu.VMEM((EB,), jnp.float32),
        pltpu.VMEM((EB,), jnp.float32),
        pltpu.VMEM((ROWS_T,), jnp.float32),
        pltpu.VMEM((ROWS_T,), jnp.float32),
        pltpu.VMEM((ROWS_T,), jnp.float32),
        pltpu.VMEM((ROWS_T,), jnp.float32),
        pltpu.VMEM_SHARED((NPAD,), jnp.float32),
        pltpu.SemaphoreType.DMA((NBUF,)),
        pltpu.SemaphoreType.DMA((NBUF,)),
    ],
)()


def kernel(x, edge_index, Wp1, bp1, Wl1, bl1, Wr1, Wp2, bp2, Wl2, bl2, Wr2):
    src = edge_index[0]
    dst = edge_index[1]
    pad = EPAD - E
    src_p = jnp.concatenate([src, jnp.zeros((pad,), jnp.int32)])
    dst_p = jnp.concatenate([dst, jnp.full((pad,), N, jnp.int32)])
    src2 = src_p.reshape(EPAD // EB, EB)
    dst2 = dst_p.reshape(EPAD // EB, EB)
    idxcat = jnp.stack([
        jnp.stack([src2 * 2, dst2], axis=1),
        jnp.stack([src2 * 2 + 1, dst2], axis=1),
    ])

    h1, xr1 = _stage1(x, Wp1, bp1, Wr1)
    ssum, cnt = _agg1(h1.reshape(2 * N, 128), idxcat)
    s2, r2 = _stage2(ssum[0, :N, :], ssum[1, :N, :], cnt[:N], xr1,
                     Wl1, bl1, Wp2, bp2, Wl2, Wr2)
    zpad = jnp.zeros((NPAD - N,), jnp.float32)
    s2_p = jnp.concatenate([s2.reshape(N), zpad])
    r2_p = jnp.concatenate([r2.reshape(N) + bl2[0], zpad])
    out = _agg2(s2_p.reshape(NPAD // 128, 128), src2, dst2, cnt, r2_p)
    return out[:N].reshape(N, 1)
